# Initial kernel scaffold; baseline (speedup 1.0000x reference)
import jax
import jax.numpy as jnp
from jax.experimental import pallas as pl


def kernel(x, coords, Wqkv_x, Wo_x, Wqkv_y, Wo_y):
    raise NotImplementedError("write your pallas kernel here")


# speedup vs baseline: 1.0201x; 1.0201x over previous
import functools

import jax
import jax.numpy as jnp
from jax.experimental import pallas as pl

_SPARSE_SHAPE = (32, 1000, 1000)
_WINDOW_SHAPE = (13, 13, 32)
_GROUP = 4096
_DIM = 128
_N = 65536
_NG = _N // _GROUP


def _window_keys(coords):
    import numpy as np
    sz, sy, sx = _SPARSE_SHAPE
    wx, wy, wz = _WINDOW_SHAPE
    max_x = int(np.ceil(sx / wx) + 1)
    max_y = int(np.ceil(sy / wy) + 1)
    max_z = int(np.ceil(sz / wz) + 1)
    m_per = max_x * max_y * max_z
    x = coords[:, 3]
    y = coords[:, 2]
    z = coords[:, 1]
    win_x = x // wx
    win_y = y // wy
    win_z = z // wz
    cix = x % wx
    ciy = y % wy
    ciz = z % wz
    bwx = coords[:, 0] * m_per + win_x * max_y * max_z + win_y * max_z + win_z
    bwy = coords[:, 0] * m_per + win_y * max_x * max_z + win_x * max_z + win_z
    wvol = wx * wy * wz
    vx = bwx * wvol + cix * (wy * wz) + ciy * wz + ciz
    vy = bwy * wvol + ciy * (wx * wz) + cix * wz + ciz
    return vx, vy


def _elu1(x):
    return jnp.where(x > 0, x + 1.0, jnp.exp(jnp.minimum(x, 0.0)))


def _group_body(x_ref, wqkv_ref, wo_ref, o_ref):
    x = x_ref[...]
    qkv = jnp.dot(x, wqkv_ref[...], preferred_element_type=jnp.float32)
    q = _elu1(qkv[:, :_DIM])
    k = _elu1(qkv[:, _DIM:2 * _DIM])
    v = qkv[:, 2 * _DIM:]
    vaug = jnp.concatenate([v, jnp.ones_like(v)], axis=1)
    s_aug = jax.lax.dot_general(k, vaug, (((0,), (0,)), ((), ())),
                                preferred_element_type=jnp.float32)
    nd = jnp.dot(q, s_aug, preferred_element_type=jnp.float32)
    out = nd[:, :_DIM] / (nd[:, _DIM:] + 1e-6)
    o_ref[...] = x + jnp.dot(out, wo_ref[...], preferred_element_type=jnp.float32)


@functools.partial(jax.jit, static_argnames=())
def _group_op(sorted_x, wqkv, wo):
    return pl.pallas_call(
        _group_body,
        grid=(_NG,),
        in_specs=[
            pl.BlockSpec((_GROUP, _DIM), lambda i: (i, 0)),
            pl.BlockSpec((_DIM, 3 * _DIM), lambda i: (0, 0)),
            pl.BlockSpec((_DIM, _DIM), lambda i: (0, 0)),
        ],
        out_specs=pl.BlockSpec((_GROUP, _DIM), lambda i: (i, 0)),
        out_shape=jax.ShapeDtypeStruct((_N, _DIM), jnp.float32),
    )(sorted_x, wqkv, wo)


def kernel(x, coords, Wqkv_x, Wo_x, Wqkv_y, Wo_y):
    coords = coords.astype(jnp.int32)
    vx, vy = _window_keys(coords)
    perm_x = jnp.argsort(vx)
    perm_y = jnp.argsort(vy)
    iota = jnp.arange(_N, dtype=jnp.int32)
    rank_x = jnp.zeros((_N,), jnp.int32).at[perm_x].set(iota)
    rank_y = jnp.zeros((_N,), jnp.int32).at[perm_y].set(iota)
    cross = rank_x[perm_y]

    sorted1 = jnp.take(x, perm_x, axis=0)
    flat1 = _group_op(sorted1, Wqkv_x, Wo_x)
    sorted2 = jnp.take(flat1, cross, axis=0)
    flat2 = _group_op(sorted2, Wqkv_y, Wo_y)
    return jnp.take(flat2, rank_y, axis=0)
